# Initial kernel scaffold; baseline (speedup 1.0000x reference)
import jax
import jax.numpy as jnp
from jax.experimental import pallas as pl


def kernel(head, Re, edge_index, bedrock_elevation, overburden_pressure, geothermal_heat_flux, ice_sliding_velocity, node_is_boundary):
    raise NotImplementedError("write your pallas kernel here")


# speedup vs baseline: 230.2634x; 230.2634x over previous
import functools

import jax
import jax.numpy as jnp
from jax import lax
from jax.experimental import pallas as pl
from jax.experimental.pallas import tpu as pltpu
from jax.experimental.pallas import tpu_sc as plsc

N_NODES = 100000
N_EDGES = 3200000
LINK_LENGTH = 100.0
GRAVITY = 9.81
WATER_DENSITY = 1000.0
ICE_DENSITY = 917.0
LATENT_HEAT = 334000.0
WATER_VISCOSITY = 1.787e-06
ICE_FLUIDITY = 6e-24
TILL_FRICTION = 0.5
FLOW_REGIME_SCALAR = 0.001
N_FP_ITERS = 15

NC, NS, L = 2, 16, 16
NW = NC * NS
EPW = N_EDGES // NW
CHUNK = 10000
NCHUNKS = EPW // CHUNK
VPC = CHUNK // L

_MESH = plsc.VectorSubcoreMesh(
    core_axis_name="c", subcore_axis_name="s", num_cores=NC, num_subcores=NS)

NR, NCL = 100, 1000
ER, ECL = 25000, 128


def _worker_id():
    return lax.axis_index("s") * NC + lax.axis_index("c")


def _make_gather(mode):
    def body(tab_hbm, src_hbm, dst_hbm, out_hbm, table, srcv, dstv, outv):
        base = _worker_id() * EPW
        pltpu.sync_copy(tab_hbm, table)

        def chunk_body(ci, _):
            off = base + ci * CHUNK
            pltpu.sync_copy(src_hbm.at[pl.ds(off, CHUNK)], srcv)
            pltpu.sync_copy(dst_hbm.at[pl.ds(off, CHUNK)], dstv)

            def vec_body(i, _):
                b = i * L
                s = srcv[pl.ds(b, L)]
                d = dstv[pl.ds(b, L)]
                ts = plsc.load_gather(table, [s])
                td = plsc.load_gather(table, [d])
                if mode == 0:
                    outv[pl.ds(b, L)] = (td - ts) / LINK_LENGTH
                else:
                    outv[pl.ds(b, L)] = 0.5 * (ts + td)
                return 0

            lax.fori_loop(0, VPC, vec_body, 0, unroll=4)
            pltpu.sync_copy(outv, out_hbm.at[pl.ds(off, CHUNK)])
            return 0

        lax.fori_loop(0, NCHUNKS, chunk_body, 0)

    return pl.kernel(
        body,
        out_type=jax.ShapeDtypeStruct((N_EDGES,), jnp.float32),
        mesh=_MESH,
        compiler_params=pltpu.CompilerParams(needs_layout_passes=False),
        scratch_types=[
            pltpu.VMEM((N_NODES,), jnp.float32),
            pltpu.VMEM((CHUNK,), jnp.int32),
            pltpu.VMEM((CHUNK,), jnp.int32),
            pltpu.VMEM((CHUNK,), jnp.float32),
        ],
    )


_gather_grad = _make_gather(0)
_gather_mean = _make_gather(1)


def _make_scatter(with_vals):
    def body(*refs):
        if with_vals:
            (src_hbm, dst_hbm, val_hbm, zero_hbm, out_hbm,
             table, srcv, dstv, valv) = refs
        else:
            (src_hbm, dst_hbm, zero_hbm, out_hbm,
             table, srcv, dstv) = refs
        wid = _worker_id()
        base = wid * EPW
        pltpu.sync_copy(zero_hbm, table)

        def chunk_body(ci, _):
            off = base + ci * CHUNK
            pltpu.sync_copy(src_hbm.at[pl.ds(off, CHUNK)], srcv)
            pltpu.sync_copy(dst_hbm.at[pl.ds(off, CHUNK)], dstv)
            if with_vals:
                pltpu.sync_copy(val_hbm.at[pl.ds(off, CHUNK)], valv)

            def vec_body(i, _):
                b = i * L
                s = srcv[pl.ds(b, L)]
                d = dstv[pl.ds(b, L)]
                if with_vals:
                    v = valv[pl.ds(b, L)]
                else:
                    v = jnp.ones((L,), jnp.float32)
                plsc.addupdate_scatter(table, [s], v)
                plsc.addupdate_scatter(table, [d], v)
                return 0

            lax.fori_loop(0, VPC, vec_body, 0, unroll=4)
            return 0

        lax.fori_loop(0, NCHUNKS, chunk_body, 0)
        pltpu.sync_copy(table, out_hbm.at[wid])

    scratch = [
        pltpu.VMEM((N_NODES,), jnp.float32),
        pltpu.VMEM((CHUNK,), jnp.int32),
        pltpu.VMEM((CHUNK,), jnp.int32),
    ]
    if with_vals:
        scratch.append(pltpu.VMEM((CHUNK,), jnp.float32))
    return pl.kernel(
        body,
        out_type=jax.ShapeDtypeStruct((NW, N_NODES), jnp.float32),
        mesh=_MESH,
        compiler_params=pltpu.CompilerParams(needs_layout_passes=False),
        scratch_types=scratch,
    )


_scatter_vals = _make_scatter(True)
_scatter_ones = _make_scatter(False)


def _node1_body(head_ref, bed_ref, ovb_ref, bnd_ref, head_o, neff_o):
    h = head_ref[...]
    b = bed_ref[...]
    ov = ovb_ref[...]
    h = jnp.where(bnd_ref[...] != 0.0, b, h)
    head_o[...] = h
    wp = WATER_DENSITY * GRAVITY * (h - b)
    ne = ov - wp
    ne = jnp.where(ne > ov, ov, ne)
    ne = jnp.where(ne < 10000.0, 10000.0, ne)
    neff_o[...] = ne


def _node1(head2, bed2, ovb2, bnd2):
    return pl.pallas_call(
        _node1_body,
        out_shape=(
            jax.ShapeDtypeStruct((NR, NCL), jnp.float32),
            jax.ShapeDtypeStruct((NR, NCL), jnp.float32),
        ),
    )(head2, bed2, ovb2, bnd2)


def _node2_body(velp_ref, degp_ref, neff_ref, geo_ref, melt_o, cond_o):
    vs = jnp.sum(velp_ref[...], axis=0)
    dg = jnp.sum(degp_ref[...], axis=0)
    sliding = vs / jnp.maximum(dg, 1.0)
    ne = neff_ref[...]
    shear = TILL_FRICTION * ne
    friction = jnp.abs(sliding * shear)
    melt = (geo_ref[...] + friction) / LATENT_HEAT
    melt_o[...] = melt
    cond_o[...] = melt / ICE_DENSITY / (ICE_FLUIDITY * (ne * ne * ne))


def _node2(velp, degp, neff2, geo2):
    return pl.pallas_call(
        _node2_body,
        out_shape=(
            jax.ShapeDtypeStruct((NR, NCL), jnp.float32),
            jax.ShapeDtypeStruct((NR, NCL), jnp.float32),
        ),
    )(velp, degp, neff2, geo2)


def _fp_body(cal_ref, grad_ref, re_ref, re_o, tr_o, di_o):
    c = cal_ref[...]
    num = c * c * c * GRAVITY
    g = grad_ref[...]
    r = re_ref[...]
    for _ in range(N_FP_ITERS):
        t = num / (12.0 * WATER_VISCOSITY * (1.0 + FLOW_REGIME_SCALAR * r))
        r = 0.5 * r + 0.5 * jnp.abs(-t * g) / WATER_VISCOSITY
    t = num / (12.0 * WATER_VISCOSITY * (1.0 + FLOW_REGIME_SCALAR * r))
    re_o[...] = r
    tr_o[...] = t
    di_o[...] = -t * g


def _fp(cal2, grad2, re2):
    grid = 25
    rows = ER // grid
    bspec = pl.BlockSpec((rows, ECL), lambda i: (i, 0))
    return pl.pallas_call(
        _fp_body,
        grid=(grid,),
        in_specs=[bspec, bspec, bspec],
        out_specs=(bspec, bspec, bspec),
        out_shape=(
            jax.ShapeDtypeStruct((ER, ECL), jnp.float32),
            jax.ShapeDtypeStruct((ER, ECL), jnp.float32),
            jax.ShapeDtypeStruct((ER, ECL), jnp.float32),
        ),
    )(cal2, grad2, re2)


def kernel(head, Re, edge_index, bedrock_elevation, overburden_pressure,
           geothermal_heat_flux, ice_sliding_velocity, node_is_boundary):
    src = edge_index[0]
    dst = edge_index[1]
    bnd2 = node_is_boundary.astype(jnp.float32).reshape(NR, NCL)

    head_p2, neff2 = _node1(
        head.reshape(NR, NCL),
        bedrock_elevation.reshape(NR, NCL),
        overburden_pressure.reshape(NR, NCL),
        bnd2,
    )
    head_p = head_p2.reshape(-1)

    grad = _gather_grad(head_p, src, dst)

    zeros_n = jnp.zeros((N_NODES,), jnp.float32)
    velp = _scatter_vals(src, dst, ice_sliding_velocity, zeros_n)
    degp = _scatter_ones(src, dst, zeros_n)

    melt2, cond2 = _node2(
        velp.reshape(NW, NR, NCL),
        degp.reshape(NW, NR, NCL),
        neff2,
        geothermal_heat_flux.reshape(NR, NCL),
    )

    cal = _gather_mean(cond2.reshape(-1), src, dst)

    re_o, tr_o, di_o = _fp(
        cal.reshape(ER, ECL), grad.reshape(ER, ECL), Re.reshape(ER, ECL))

    return (
        head_p,
        grad,
        neff2.reshape(-1),
        melt2.reshape(-1),
        cond2.reshape(-1),
        re_o.reshape(-1),
        tr_o.reshape(-1),
        di_o.reshape(-1),
    )


# speedup vs baseline: 472.4526x; 2.0518x over previous
import functools

import jax
import jax.numpy as jnp
from jax import lax
from jax.experimental import pallas as pl
from jax.experimental.pallas import tpu as pltpu
from jax.experimental.pallas import tpu_sc as plsc

N_NODES = 100000
N_EDGES = 3200000
LINK_LENGTH = 100.0
GRAVITY = 9.81
WATER_DENSITY = 1000.0
ICE_DENSITY = 917.0
LATENT_HEAT = 334000.0
WATER_VISCOSITY = 1.787e-06
ICE_FLUIDITY = 6e-24
TILL_FRICTION = 0.5
FLOW_REGIME_SCALAR = 0.001
N_FP_ITERS = 15

NC, NS, L = 2, 16, 16
NW = NC * NS
EPW = N_EDGES // NW
CHUNK = 4000
NCHUNKS = EPW // CHUNK
UNROLL = 5

_MESH = plsc.VectorSubcoreMesh(
    core_axis_name="c", subcore_axis_name="s", num_cores=NC, num_subcores=NS)
_SC_PARAMS = pltpu.CompilerParams(needs_layout_passes=False)

NR, NCL = 100, 1000
ER, ECL = 25000, 128


def _worker_id():
    return lax.axis_index("s") * NC + lax.axis_index("c")


def _make_gather(mode):
    def body(tab_hbm, src_hbm, dst_hbm, out_hbm, table,
             srcv0, dstv0, outv0, srcv1, dstv1, outv1,
             tsem, isem0, isem1, osem0, osem1):
        base = _worker_id() * EPW
        bufs = ((srcv0, dstv0, outv0, isem0, osem0),
                (srcv1, dstv1, outv1, isem1, osem1))

        table_cp = pltpu.async_copy(tab_hbm, table, tsem)

        def start_in(ci):
            s, d, _, isem, _ = bufs[ci % 2]
            off = base + ci * CHUNK
            c1 = pltpu.async_copy(src_hbm.at[pl.ds(off, CHUNK)], s, isem)
            c2 = pltpu.async_copy(dst_hbm.at[pl.ds(off, CHUNK)], d, isem)
            return (c1, c2)

        in_cp = {0: start_in(0)}
        out_cp = {}
        for ci in range(NCHUNKS):
            s, d, o, isem, osem = bufs[ci % 2]
            if ci + 1 < NCHUNKS:
                in_cp[ci + 1] = start_in(ci + 1)
            for cp in in_cp.pop(ci):
                cp.wait()
            if ci == 0:
                table_cp.wait()
            if ci >= 2:
                out_cp.pop(ci - 2).wait()

            @plsc.parallel_loop(0, CHUNK, step=L, unroll=UNROLL)
            def _(i, _s=s, _d=d, _o=o):
                sv = _s[pl.ds(i, L)]
                dv = _d[pl.ds(i, L)]
                ts = plsc.load_gather(table, [sv])
                td = plsc.load_gather(table, [dv])
                if mode == 0:
                    _o[pl.ds(i, L)] = (td - ts) / LINK_LENGTH
                else:
                    _o[pl.ds(i, L)] = 0.5 * (ts + td)

            out_cp[ci] = pltpu.async_copy(
                o, out_hbm.at[pl.ds(base + ci * CHUNK, CHUNK)], osem)
        for cp in out_cp.values():
            cp.wait()

    return pl.kernel(
        body,
        out_type=jax.ShapeDtypeStruct((N_EDGES,), jnp.float32),
        mesh=_MESH,
        compiler_params=_SC_PARAMS,
        scratch_types=[
            pltpu.VMEM((N_NODES,), jnp.float32),
            pltpu.VMEM((CHUNK,), jnp.int32),
            pltpu.VMEM((CHUNK,), jnp.int32),
            pltpu.VMEM((CHUNK,), jnp.float32),
            pltpu.VMEM((CHUNK,), jnp.int32),
            pltpu.VMEM((CHUNK,), jnp.int32),
            pltpu.VMEM((CHUNK,), jnp.float32),
            pltpu.SemaphoreType.DMA,
            pltpu.SemaphoreType.DMA,
            pltpu.SemaphoreType.DMA,
            pltpu.SemaphoreType.DMA,
            pltpu.SemaphoreType.DMA,
        ],
    )


_gather_grad = _make_gather(0)
_gather_mean = _make_gather(1)


def _make_scatter(with_vals):
    def body(*refs):
        if with_vals:
            (src_hbm, dst_hbm, val_hbm, out_hbm, table,
             srcv0, dstv0, valv0, srcv1, dstv1, valv1, isem0, isem1) = refs
        else:
            (src_hbm, dst_hbm, out_hbm, table,
             srcv0, dstv0, srcv1, dstv1, isem0, isem1) = refs
            valv0 = valv1 = None
        wid = _worker_id()
        base = wid * EPW
        bufs = ((srcv0, dstv0, valv0, isem0),
                (srcv1, dstv1, valv1, isem1))

        def start_in(ci):
            s, d, v, isem = bufs[ci % 2]
            off = base + ci * CHUNK
            cps = [pltpu.async_copy(src_hbm.at[pl.ds(off, CHUNK)], s, isem),
                   pltpu.async_copy(dst_hbm.at[pl.ds(off, CHUNK)], d, isem)]
            if with_vals:
                cps.append(
                    pltpu.async_copy(val_hbm.at[pl.ds(off, CHUNK)], v, isem))
            return cps

        in_cp = {0: start_in(0)}

        @plsc.parallel_loop(0, N_NODES, step=L, unroll=UNROLL)
        def _(i):
            table[pl.ds(i, L)] = jnp.zeros((L,), jnp.float32)

        for ci in range(NCHUNKS):
            s, d, v, isem = bufs[ci % 2]
            if ci + 1 < NCHUNKS:
                in_cp[ci + 1] = start_in(ci + 1)
            for cp in in_cp.pop(ci):
                cp.wait()

            @plsc.parallel_loop(0, CHUNK, step=L, unroll=UNROLL)
            def _(i, _s=s, _d=d, _v=v):
                sv = _s[pl.ds(i, L)]
                dv = _d[pl.ds(i, L)]
                if with_vals:
                    vv = _v[pl.ds(i, L)]
                else:
                    vv = jnp.ones((L,), jnp.float32)
                plsc.addupdate_scatter(table, [sv], vv)
                plsc.addupdate_scatter(table, [dv], vv)

        pltpu.sync_copy(table, out_hbm.at[wid])

    scratch = [pltpu.VMEM((N_NODES,), jnp.float32)]
    per_buf = [pltpu.VMEM((CHUNK,), jnp.int32), pltpu.VMEM((CHUNK,), jnp.int32)]
    if with_vals:
        per_buf.append(pltpu.VMEM((CHUNK,), jnp.float32))
    scratch += per_buf + per_buf + [pltpu.SemaphoreType.DMA,
                                    pltpu.SemaphoreType.DMA]
    return pl.kernel(
        body,
        out_type=jax.ShapeDtypeStruct((NW, N_NODES), jnp.float32),
        mesh=_MESH,
        compiler_params=_SC_PARAMS,
        scratch_types=scratch,
    )


_scatter_vals = _make_scatter(True)
_scatter_ones = _make_scatter(False)


def _node1_body(head_ref, bed_ref, ovb_ref, bnd_ref, head_o, neff_o):
    h = head_ref[...]
    b = bed_ref[...]
    ov = ovb_ref[...]
    h = jnp.where(bnd_ref[...] != 0.0, b, h)
    head_o[...] = h
    wp = WATER_DENSITY * GRAVITY * (h - b)
    ne = ov - wp
    ne = jnp.where(ne > ov, ov, ne)
    ne = jnp.where(ne < 10000.0, 10000.0, ne)
    neff_o[...] = ne


def _node1(head2, bed2, ovb2, bnd2):
    return pl.pallas_call(
        _node1_body,
        out_shape=(
            jax.ShapeDtypeStruct((NR, NCL), jnp.float32),
            jax.ShapeDtypeStruct((NR, NCL), jnp.float32),
        ),
    )(head2, bed2, ovb2, bnd2)


def _node2_body(velp_ref, degp_ref, neff_ref, geo_ref, melt_o, cond_o):
    vs = jnp.sum(velp_ref[...], axis=0)
    dg = jnp.sum(degp_ref[...], axis=0)
    sliding = vs / jnp.maximum(dg, 1.0)
    ne = neff_ref[...]
    shear = TILL_FRICTION * ne
    friction = jnp.abs(sliding * shear)
    melt = (geo_ref[...] + friction) / LATENT_HEAT
    melt_o[...] = melt
    cond_o[...] = melt / ICE_DENSITY / (ICE_FLUIDITY * (ne * ne * ne))


def _node2(velp, degp, neff2, geo2):
    return pl.pallas_call(
        _node2_body,
        out_shape=(
            jax.ShapeDtypeStruct((NR, NCL), jnp.float32),
            jax.ShapeDtypeStruct((NR, NCL), jnp.float32),
        ),
    )(velp, degp, neff2, geo2)


def _fp_body(cal_ref, grad_ref, re_ref, re_o, tr_o, di_o):
    c = cal_ref[...]
    num = c * c * c * GRAVITY
    g = grad_ref[...]
    r = re_ref[...]
    k = jnp.abs(num * g) * (1.0 / (24.0 * WATER_VISCOSITY * WATER_VISCOSITY))
    for _ in range(N_FP_ITERS):
        r = 0.5 * r + k / (1.0 + FLOW_REGIME_SCALAR * r)
    t = num / (12.0 * WATER_VISCOSITY * (1.0 + FLOW_REGIME_SCALAR * r))
    re_o[...] = r
    tr_o[...] = t
    di_o[...] = -t * g


def _fp(cal2, grad2, re2):
    grid = 25
    rows = ER // grid
    bspec = pl.BlockSpec((rows, ECL), lambda i: (i, 0))
    return pl.pallas_call(
        _fp_body,
        grid=(grid,),
        in_specs=[bspec, bspec, bspec],
        out_specs=(bspec, bspec, bspec),
        out_shape=(
            jax.ShapeDtypeStruct((ER, ECL), jnp.float32),
            jax.ShapeDtypeStruct((ER, ECL), jnp.float32),
            jax.ShapeDtypeStruct((ER, ECL), jnp.float32),
        ),
    )(cal2, grad2, re2)


def kernel(head, Re, edge_index, bedrock_elevation, overburden_pressure,
           geothermal_heat_flux, ice_sliding_velocity, node_is_boundary):
    src = edge_index[0]
    dst = edge_index[1]
    bnd2 = node_is_boundary.astype(jnp.float32).reshape(NR, NCL)

    head_p2, neff2 = _node1(
        head.reshape(NR, NCL),
        bedrock_elevation.reshape(NR, NCL),
        overburden_pressure.reshape(NR, NCL),
        bnd2,
    )
    head_p = head_p2.reshape(-1)

    grad = _gather_grad(head_p, src, dst)

    velp = _scatter_vals(src, dst, ice_sliding_velocity)
    degp = _scatter_ones(src, dst)

    melt2, cond2 = _node2(
        velp.reshape(NW, NR, NCL),
        degp.reshape(NW, NR, NCL),
        neff2,
        geothermal_heat_flux.reshape(NR, NCL),
    )

    cal = _gather_mean(cond2.reshape(-1), src, dst)

    re_o, tr_o, di_o = _fp(
        cal.reshape(ER, ECL), grad.reshape(ER, ECL), Re.reshape(ER, ECL))

    return (
        head_p,
        grad,
        neff2.reshape(-1),
        melt2.reshape(-1),
        cond2.reshape(-1),
        re_o.reshape(-1),
        tr_o.reshape(-1),
        di_o.reshape(-1),
    )
